# baseline (speedup 1.0000x reference)
import functools

import jax
import jax.numpy as jnp
from jax import lax
from jax.experimental import pallas as pl
from jax.experimental.pallas import tpu as pltpu
from jax.experimental.pallas import tpu_sc as plsc

B = 16384
NUM_TABLES = 100000
META_IN, META_OUT = 16, 32
EMB_IN, EMB_OUT = 128, 64
FINAL_IN = META_OUT + EMB_OUT
SIZE = 64

NC, NS = 2, 16
NW = NC * NS
BPW = B // NW
CHUNK = 128
NCHUNK = BPW // CHUNK

@functools.cache
def _make_sc_gathers():
    mesh = plsc.VectorSubcoreMesh(core_axis_name="c", subcore_axis_name="s")

    @functools.partial(
        pl.kernel,
        out_type=(
            jax.ShapeDtypeStruct((B, EMB_IN), jnp.float32),
            jax.ShapeDtypeStruct((2, B, 8), jnp.float32),
        ),
        mesh=mesh,
        scratch_types=[
            pltpu.VMEM((BPW,), jnp.int32),
            pltpu.VMEM((2, BPW, 8), jnp.float32),
            pltpu.VMEM((BPW, EMB_IN), jnp.float32),
            [pltpu.SemaphoreType.DMA] * NCHUNK,
            pltpu.SemaphoreType.DMA,
            pltpu.SemaphoreType.DMA,
        ],
        compiler_params=pltpu.CompilerParams(
            use_tc_tiling_on_sc=False, needs_layout_passes=False),
    )
    def _gather(idx_hbm, etab_hbm, mt_hbm, emb_out, meta_out,
                idx_v, gbuf, erows_v, esems, msem, wsem):
        wid = lax.axis_index("s") * NC + lax.axis_index("c")
        base = wid * BPW
        pltpu.sync_copy(idx_hbm.at[wid], idx_v)
        ecopies = []
        for j in range(NCHUNK):
            ecopies.append(pltpu.async_copy(
                etab_hbm.at[idx_v.at[pl.ds(j * CHUNK, CHUNK)]],
                erows_v.at[pl.ds(j * CHUNK, CHUNK)], esems[j]))
        mcopies = []
        for p in range(2):
            mcopies.append(pltpu.async_copy(
                mt_hbm.at[p].at[idx_v], gbuf.at[p], msem))
        wcopies = []
        for j in range(NCHUNK):
            ecopies[j].wait()
            wcopies.append(pltpu.async_copy(
                erows_v.at[pl.ds(j * CHUNK, CHUNK)],
                emb_out.at[pl.ds(base + j * CHUNK, CHUNK)], wsem))
        for p in range(2):
            mcopies[p].wait()
            wcopies.append(pltpu.async_copy(
                gbuf.at[p], meta_out.at[p, pl.ds(base, BPW), :], wsem))
        for w in wcopies:
            w.wait()

    return _gather


def _gelu(x):
    return 0.5 * x * (1.0 + lax.erf(x * 0.7071067811865476))


def _head_body(mfT_ref, ef_ref, wmT_ref, bm_ref, weT_ref, be_ref,
               g_ref, bln_ref, wfT_ref, bf_ref, o_ref):
    mfT = mfT_ref[...]
    ef = ef_ref[...]
    metaT = _gelu(jnp.dot(wmT_ref[...], mfT,
                          preferred_element_type=jnp.float32)
                  + bm_ref[...])
    embT = (lax.dot_general(weT_ref[...], ef, (((1,), (1,)), ((), ())),
                            preferred_element_type=jnp.float32)
            + be_ref[...])
    s = jnp.sum(metaT, axis=0, keepdims=True) + jnp.sum(embT, axis=0,
                                                        keepdims=True)
    ss = (jnp.sum(metaT * metaT, axis=0, keepdims=True)
          + jnp.sum(embT * embT, axis=0, keepdims=True))
    mu = s * (1.0 / FINAL_IN)
    var = ss * (1.0 / FINAL_IN) - mu * mu
    inv = lax.rsqrt(var + 1e-5)
    g = g_ref[...]
    bln = bln_ref[...]
    meta_n = (metaT - mu) * inv * g[:META_OUT, :] + bln[:META_OUT, :]
    emb_n = (embT - mu) * inv * g[META_OUT:, :] + bln[META_OUT:, :]
    h = jnp.concatenate([meta_n, emb_n], axis=0)
    out = (jnp.dot(wfT_ref[...], h, preferred_element_type=jnp.float32)
           + bf_ref[...])
    o_ref[...] = _gelu(out)


BM = 2048


def _head(meta_featT, emb_feat, W_meta, b_meta, W_embed, b_embed,
          ln_g, ln_b, W_final, b_final):
    full = lambda shape: pl.BlockSpec(shape, lambda i: (0,) * len(shape))
    outT = pl.pallas_call(
        _head_body,
        grid=(B // BM,),
        in_specs=[
            pl.BlockSpec((META_IN, BM), lambda i: (0, i)),
            pl.BlockSpec((BM, EMB_IN), lambda i: (i, 0)),
            full((META_OUT, META_IN)),
            full((META_OUT, 1)),
            full((EMB_OUT, EMB_IN)),
            full((EMB_OUT, 1)),
            full((FINAL_IN, 1)),
            full((FINAL_IN, 1)),
            full((SIZE, FINAL_IN)),
            full((SIZE, 1)),
        ],
        out_specs=pl.BlockSpec((SIZE, BM), lambda i: (0, i)),
        out_shape=jax.ShapeDtypeStruct((SIZE, B), jnp.float32),
        compiler_params=pltpu.CompilerParams(
            dimension_semantics=("arbitrary",)),
    )(meta_featT, emb_feat, W_meta.T, b_meta.reshape(-1, 1), W_embed.T,
      b_embed.reshape(-1, 1), ln_g.reshape(-1, 1), ln_b.reshape(-1, 1),
      W_final.T, b_final.reshape(-1, 1))
    return outT.T


def kernel(table_idx, meta_table, embed_table, W_meta, b_meta,
           W_embed, b_embed, ln_g, ln_b, W_final, b_final):
    idx = table_idx.astype(jnp.int32).reshape(NW, BPW)
    mt8 = meta_table.T.reshape(2, 8, NUM_TABLES).transpose(0, 2, 1)
    emb_feat, meta_g = _make_sc_gathers()(idx, embed_table, mt8)
    meta_feat = meta_g.transpose(0, 2, 1).reshape(META_IN, B)
    return _head(meta_feat, emb_feat, W_meta, b_meta, W_embed,
                 b_embed, ln_g, ln_b, W_final, b_final)


# speedup vs baseline: 2.7292x; 2.7292x over previous
import functools

import jax
import jax.numpy as jnp
from jax import lax
from jax.experimental import pallas as pl
from jax.experimental.pallas import tpu as pltpu
from jax.experimental.pallas import tpu_sc as plsc

B = 16384
NUM_TABLES = 100000
META_IN, META_OUT = 16, 32
EMB_IN, EMB_OUT = 128, 64
FINAL_IN = META_OUT + EMB_OUT
SIZE = 64

NC, NS = 2, 16
NW = NC * NS
BPW = B // NW
CHUNK = 128
NCHUNK = BPW // CHUNK

@functools.cache
def _make_sc_gathers():
    mesh = plsc.VectorSubcoreMesh(core_axis_name="c", subcore_axis_name="s")

    @functools.partial(
        pl.kernel,
        out_type=(
            jax.ShapeDtypeStruct((B, EMB_IN), jnp.float32),
            jax.ShapeDtypeStruct((META_IN, B), jnp.float32),
        ),
        mesh=mesh,
        scratch_types=[
            pltpu.VMEM((BPW,), jnp.int32),
            pltpu.VMEM((META_IN, BPW), jnp.float32),
            pltpu.VMEM((BPW, EMB_IN), jnp.float32),
            [pltpu.SemaphoreType.DMA] * NCHUNK,
            pltpu.SemaphoreType.DMA,
            pltpu.SemaphoreType.DMA,
        ],
        compiler_params=pltpu.CompilerParams(
            use_tc_tiling_on_sc=False, needs_layout_passes=False),
    )
    def _gather(idx_hbm, etab_hbm, mt_hbm, emb_out, meta_out,
                idx_v, gbuf, erows_v, esems, msem, wsem):
        wid = lax.axis_index("s") * NC + lax.axis_index("c")
        base = wid * BPW
        pltpu.sync_copy(idx_hbm.at[wid], idx_v)
        ecopies = []
        for j in range(NCHUNK):
            ecopies.append(pltpu.async_copy(
                etab_hbm.at[idx_v.at[pl.ds(j * CHUNK, CHUNK)]],
                erows_v.at[pl.ds(j * CHUNK, CHUNK)], esems[j]))
        mcopies = []
        for f in range(META_IN):
            mcopies.append(pltpu.async_copy(
                mt_hbm.at[f].at[idx_v], gbuf.at[f], msem))
        wcopies = []
        for j in range(NCHUNK):
            ecopies[j].wait()
            wcopies.append(pltpu.async_copy(
                erows_v.at[pl.ds(j * CHUNK, CHUNK)],
                emb_out.at[pl.ds(base + j * CHUNK, CHUNK)], wsem))
        for f in range(META_IN):
            mcopies[f].wait()
            wcopies.append(pltpu.async_copy(
                gbuf.at[f], meta_out.at[f, pl.ds(base, BPW)], wsem))
        for w in wcopies:
            w.wait()

    return _gather


def _gelu(x):
    return 0.5 * x * (1.0 + lax.erf(x * 0.7071067811865476))


def _head_body(mfT_ref, ef_ref, wmT_ref, bm_ref, weT_ref, be_ref,
               g_ref, bln_ref, wfT_ref, bf_ref, o_ref):
    mfT = mfT_ref[...]
    ef = ef_ref[...]
    metaT = _gelu(jnp.dot(wmT_ref[...], mfT,
                          preferred_element_type=jnp.float32)
                  + bm_ref[...])
    embT = (lax.dot_general(weT_ref[...], ef, (((1,), (1,)), ((), ())),
                            preferred_element_type=jnp.float32)
            + be_ref[...])
    s = jnp.sum(metaT, axis=0, keepdims=True) + jnp.sum(embT, axis=0,
                                                        keepdims=True)
    ss = (jnp.sum(metaT * metaT, axis=0, keepdims=True)
          + jnp.sum(embT * embT, axis=0, keepdims=True))
    mu = s * (1.0 / FINAL_IN)
    var = ss * (1.0 / FINAL_IN) - mu * mu
    inv = lax.rsqrt(var + 1e-5)
    g = g_ref[...]
    bln = bln_ref[...]
    meta_n = (metaT - mu) * inv * g[:META_OUT, :] + bln[:META_OUT, :]
    emb_n = (embT - mu) * inv * g[META_OUT:, :] + bln[META_OUT:, :]
    h = jnp.concatenate([meta_n, emb_n], axis=0)
    out = (jnp.dot(wfT_ref[...], h, preferred_element_type=jnp.float32)
           + bf_ref[...])
    o_ref[...] = _gelu(out)


BM = 4096


def _head(meta_featT, emb_feat, W_meta, b_meta, W_embed, b_embed,
          ln_g, ln_b, W_final, b_final):
    full = lambda shape: pl.BlockSpec(shape, lambda i: (0,) * len(shape))
    outT = pl.pallas_call(
        _head_body,
        grid=(B // BM,),
        in_specs=[
            pl.BlockSpec((META_IN, BM), lambda i: (0, i)),
            pl.BlockSpec((BM, EMB_IN), lambda i: (i, 0)),
            full((META_OUT, META_IN)),
            full((META_OUT, 1)),
            full((EMB_OUT, EMB_IN)),
            full((EMB_OUT, 1)),
            full((FINAL_IN, 1)),
            full((FINAL_IN, 1)),
            full((SIZE, FINAL_IN)),
            full((SIZE, 1)),
        ],
        out_specs=pl.BlockSpec((SIZE, BM), lambda i: (0, i)),
        out_shape=jax.ShapeDtypeStruct((SIZE, B), jnp.float32),
        compiler_params=pltpu.CompilerParams(
            dimension_semantics=("arbitrary",)),
    )(meta_featT, emb_feat, W_meta.T, b_meta.reshape(-1, 1), W_embed.T,
      b_embed.reshape(-1, 1), ln_g.reshape(-1, 1), ln_b.reshape(-1, 1),
      W_final.T, b_final.reshape(-1, 1))
    return outT.T


def kernel(table_idx, meta_table, embed_table, W_meta, b_meta,
           W_embed, b_embed, ln_g, ln_b, W_final, b_final):
    idx = table_idx.astype(jnp.int32).reshape(NW, BPW)
    emb_feat, meta_feat = _make_sc_gathers()(idx, embed_table, meta_table.T)
    return _head(meta_feat, emb_feat, W_meta, b_meta, W_embed,
                 b_embed, ln_g, ln_b, W_final, b_final)


# speedup vs baseline: 2.7529x; 1.0087x over previous
import functools

import jax
import jax.numpy as jnp
from jax import lax
from jax.experimental import pallas as pl
from jax.experimental.pallas import tpu as pltpu
from jax.experimental.pallas import tpu_sc as plsc

B = 16384
NUM_TABLES = 100000
META_IN, META_OUT = 16, 32
EMB_IN, EMB_OUT = 128, 64
FINAL_IN = META_OUT + EMB_OUT
SIZE = 64

NC, NS = 2, 16
NW = NC * NS
BPW = B // NW
CHUNK = 128
NCHUNK = BPW // CHUNK

@functools.cache
def _make_sc_gathers():
    mesh = plsc.VectorSubcoreMesh(core_axis_name="c", subcore_axis_name="s")

    @functools.partial(
        pl.kernel,
        out_type=(
            jax.ShapeDtypeStruct((B, EMB_IN), jnp.float32),
            jax.ShapeDtypeStruct((META_IN, B), jnp.float32),
        ),
        mesh=mesh,
        scratch_types=[
            pltpu.VMEM((BPW,), jnp.int32),
            pltpu.VMEM((META_IN, BPW), jnp.float32),
            pltpu.VMEM((BPW, EMB_IN), jnp.float32),
            [pltpu.SemaphoreType.DMA] * NCHUNK,
            pltpu.SemaphoreType.DMA,
            pltpu.SemaphoreType.DMA,
        ],
        compiler_params=pltpu.CompilerParams(
            use_tc_tiling_on_sc=False, needs_layout_passes=False),
    )
    def _gather(idx_hbm, etab_hbm, mt_hbm, emb_out, meta_out,
                idx_v, gbuf, erows_v, esems, msem, wsem):
        wid = lax.axis_index("s") * NC + lax.axis_index("c")
        base = wid * BPW
        pltpu.sync_copy(idx_hbm.at[wid], idx_v)
        ecopies = []
        for j in range(NCHUNK):
            ecopies.append(pltpu.async_copy(
                etab_hbm.at[idx_v.at[pl.ds(j * CHUNK, CHUNK)]],
                erows_v.at[pl.ds(j * CHUNK, CHUNK)], esems[j]))
        mcopies = []
        for f in range(META_IN):
            mcopies.append(pltpu.async_copy(
                mt_hbm.at[f].at[idx_v], gbuf.at[f], msem))
        wcopies = []
        for j in range(NCHUNK):
            ecopies[j].wait()
            wcopies.append(pltpu.async_copy(
                erows_v.at[pl.ds(j * CHUNK, CHUNK)],
                emb_out.at[pl.ds(base + j * CHUNK, CHUNK)], wsem))
        for f in range(META_IN):
            mcopies[f].wait()
            wcopies.append(pltpu.async_copy(
                gbuf.at[f], meta_out.at[f, pl.ds(base, BPW)], wsem))
        for w in wcopies:
            w.wait()

    return _gather


def _gelu(x):
    return 0.5 * x * (1.0 + lax.erf(x * 0.7071067811865476))


def _head_body(mfT_ref, ef_ref, wmT_ref, bm_ref, weT_ref, be_ref,
               g_ref, bln_ref, wfT_ref, bf_ref, o_ref):
    mfT = mfT_ref[...]
    ef = ef_ref[...]
    metaT = _gelu(jnp.dot(wmT_ref[...], mfT,
                          preferred_element_type=jnp.float32)
                  + bm_ref[...])
    embT = (lax.dot_general(weT_ref[...], ef, (((1,), (1,)), ((), ())),
                            preferred_element_type=jnp.float32)
            + be_ref[...])
    s = jnp.sum(metaT, axis=0, keepdims=True) + jnp.sum(embT, axis=0,
                                                        keepdims=True)
    ss = (jnp.sum(metaT * metaT, axis=0, keepdims=True)
          + jnp.sum(embT * embT, axis=0, keepdims=True))
    mu = s * (1.0 / FINAL_IN)
    var = ss * (1.0 / FINAL_IN) - mu * mu
    inv = lax.rsqrt(var + 1e-5)
    g = g_ref[...]
    bln = bln_ref[...]
    meta_n = (metaT - mu) * inv * g[:META_OUT, :] + bln[:META_OUT, :]
    emb_n = (embT - mu) * inv * g[META_OUT:, :] + bln[META_OUT:, :]
    h = jnp.concatenate([meta_n, emb_n], axis=0)
    out = (jnp.dot(wfT_ref[...], h, preferred_element_type=jnp.float32)
           + bf_ref[...])
    o_ref[...] = _gelu(out)


BM = 8192


def _head(meta_featT, emb_feat, W_meta, b_meta, W_embed, b_embed,
          ln_g, ln_b, W_final, b_final):
    full = lambda shape: pl.BlockSpec(shape, lambda i: (0,) * len(shape))
    outT = pl.pallas_call(
        _head_body,
        grid=(B // BM,),
        in_specs=[
            pl.BlockSpec((META_IN, BM), lambda i: (0, i)),
            pl.BlockSpec((BM, EMB_IN), lambda i: (i, 0)),
            full((META_OUT, META_IN)),
            full((META_OUT, 1)),
            full((EMB_OUT, EMB_IN)),
            full((EMB_OUT, 1)),
            full((FINAL_IN, 1)),
            full((FINAL_IN, 1)),
            full((SIZE, FINAL_IN)),
            full((SIZE, 1)),
        ],
        out_specs=pl.BlockSpec((SIZE, BM), lambda i: (0, i)),
        out_shape=jax.ShapeDtypeStruct((SIZE, B), jnp.float32),
        compiler_params=pltpu.CompilerParams(
            dimension_semantics=("arbitrary",)),
    )(meta_featT, emb_feat, W_meta.T, b_meta.reshape(-1, 1), W_embed.T,
      b_embed.reshape(-1, 1), ln_g.reshape(-1, 1), ln_b.reshape(-1, 1),
      W_final.T, b_final.reshape(-1, 1))
    return outT.T


def kernel(table_idx, meta_table, embed_table, W_meta, b_meta,
           W_embed, b_embed, ln_g, ln_b, W_final, b_final):
    idx = table_idx.astype(jnp.int32).reshape(NW, BPW)
    emb_feat, meta_feat = _make_sc_gathers()(idx, embed_table, meta_table.T)
    return _head(meta_feat, emb_feat, W_meta, b_meta, W_embed,
                 b_embed, ln_g, ln_b, W_final, b_final)
